# baseline (speedup 1.0000x reference)
import functools

import jax
import jax.numpy as jnp
from jax import lax
from jax.experimental import pallas as pl
from jax.experimental.pallas import tpu as pltpu
from jax.experimental.pallas import tpu_sc as plsc

T = 2048
E = 128
H = 8
HE = H * E
KG = 4
VS = 24
VSP = 32
RB = 256
MIN_SIGMA, SIGMA_SCALE, MMULT = 0.05, 0.1, 3.0
SIGMA_BOOST, EPS = 2.0, 1e-7
NW = 32
RPW = T // NW


def _dot16(a, b):
    return jnp.dot(a.astype(jnp.bfloat16), b.astype(jnp.bfloat16),
                   preferred_element_type=jnp.float32)


def _softplus(v):
    return jnp.maximum(v, 0.0) + jnp.log1p(jnp.exp(-jnp.abs(v)))


def _front(x_ref, wq_ref, wk_ref, wv_ref, wp1a_ref, wp1b_ref, bp1_ref,
           wp2_ref, bp2_ref, glob_ref, locr_ref,
           q_ref, kv_ref, iv_ref, w_ref):
    blk = pl.program_id(0)
    xb = x_ref[...]
    rows_f = (blk * RB + lax.broadcasted_iota(jnp.int32, (RB, 1), 0)).astype(jnp.float32)
    coords = rows_f * (1.0 / T)
    h1 = _dot16(xb, wp1a_ref[...])
    cterm = (coords.astype(jnp.bfloat16).astype(jnp.float32)
             * wp1b_ref[...].astype(jnp.bfloat16).astype(jnp.float32))
    h1 = jnp.maximum(h1 + cterm + bp1_ref[...], 0.0)
    params = _dot16(h1, wp2_ref[...])
    params = params + bp2_ref[...]
    means = jnp.clip(rows_f - MMULT * _softplus(params[:, :KG]), 0.0, T - 1.0)
    sig = (_softplus(params[:, KG:] + SIGMA_BOOST) + MIN_SIGMA) * (T * SIGMA_SCALE)
    fl = jnp.floor(means)

    parts = []
    for g in range(KG):
        fg = fl[:, g:g + 1]
        parts += [fg, fg + 1.0,
                  glob_ref[:, 2 * g:2 * g + 1], glob_ref[:, 2 * g + 1:2 * g + 2],
                  fg - 32.0 + locr_ref[:, 2 * g:2 * g + 1],
                  fg - 32.0 + locr_ref[:, 2 * g + 1:2 * g + 2]]
    idxf = jnp.clip(jnp.concatenate(parts, axis=1), 0.0, T - 1.0)
    ivi = idxf.astype(jnp.int32)

    lane = lax.broadcasted_iota(jnp.int32, (RB, VS), 1)
    dup = jnp.zeros((RB, VS), jnp.bool_)
    for jp in range(VS - 1):
        dup = dup | ((ivi == ivi[:, jp:jp + 1]) & (lane > jp))
    rows_i = blk * RB + lax.broadcasted_iota(jnp.int32, (RB, VS), 0)
    masked = dup | (ivi > rows_i)

    wacc = jnp.zeros((RB, VS), jnp.float32)
    for g in range(KG):
        diff = idxf - means[:, g:g + 1]
        pr = jnp.exp(-0.5 * diff * diff / (sig[:, g:g + 1] + EPS))
        pr = jnp.where(masked, 0.0, pr)
        wacc = wacc + pr / jnp.sum(pr, axis=1, keepdims=True)

    q_ref[...] = _dot16(xb, wq_ref[...]) * (E ** -0.5)

    def _pack2(m):
        mb = m.astype(jnp.bfloat16)
        lo = lax.bitcast_convert_type(mb[:, :HE // 2], jnp.uint16).astype(jnp.uint32)
        hi = lax.bitcast_convert_type(mb[:, HE // 2:], jnp.uint16).astype(jnp.uint32)
        return lax.bitcast_convert_type(lo | (hi << 16), jnp.float32)

    kv_ref[:, :HE // 2] = _pack2(_dot16(xb, wk_ref[...]))
    kv_ref[:, HE // 2:] = _pack2(_dot16(xb, wv_ref[...]))
    iv_ref[...] = jnp.concatenate(
        [ivi, jnp.zeros((RB, VSP - VS), jnp.int32)], axis=1)
    w_ref[...] = jnp.concatenate(
        [wacc, jnp.zeros((RB, VSP - VS), jnp.float32)], axis=1)


def _outproj(o_ref, wu_ref, bu_ref, y_ref):
    y_ref[...] = _dot16(o_ref[...], wu_ref[...]) + bu_ref[...]


def _sc_body(q_hbm, kv_hbm, iv_hbm, w_hbm, out_hbm,
             iv_all, w_all, qa, qb, kva, kvb, mat_v, p_v, erep_v, oa, ob,
             ska, skb, sqa, sqb, soa, sob):
    wid = lax.axis_index("s") * 2 + lax.axis_index("c")
    base = wid * RPW

    pltpu.sync_copy(iv_hbm.at[pl.ds(base * VSP, RPW * VSP)], iv_all)
    pltpu.sync_copy(w_hbm.at[pl.ds(base * VSP, RPW * VSP)], w_all)

    neg_inf = jnp.float32(-jnp.inf)
    lane16 = lax.iota(jnp.int32, 16)
    zero16 = jnp.zeros((16,), jnp.float32)
    j0 = lane16
    j1 = jnp.minimum(lane16 + 16, VS - 1)

    def gather_copy(rl, kvx, sem):
        return pltpu.make_async_copy(
            kv_hbm.at[iv_all.at[pl.ds(rl * VSP, VS)]], kvx, sem)

    def q_copy(rl, qx, sem):
        return pltpu.make_async_copy(
            q_hbm.at[pl.ds((base + rl) * HE, HE)], qx, sem)

    def out_copy(rl, ox, sem):
        return pltpu.make_async_copy(
            ox, out_hbm.at[pl.ds((base + rl) * HE, HE)], sem)

    def start_row(rl, kvx, qx, skx, sqx):
        gather_copy(rl, kvx, skx).start()
        q_copy(rl, qx, sqx).start()

    jb0 = j0 * 17
    jb1 = j1 * 17

    MOFF = VS * 17

    def _unpack2(v16):
        return plsc.unpack(plsc.bitcast(v16, jnp.bfloat16),
                           format=plsc.PackFormat.INTERLEAVED)

    def compute_row(rl, kvx, qx, ox):
        w0 = w_all[pl.ds(rl * VSP, 16)]
        w1 = w_all[pl.ds(rl * VSP + 16, 16)]
        for hp in range(H // 2):
            hb = hp * E
            ha = (hp + 4) * E
            qlo = [qx[pl.ds(hb + c * 16, 16)] for c in range(8)]
            qhi = [qx[pl.ds(ha + c * 16, 16)] for c in range(8)]

            def dbody(j, _):
                pa, pb = [], []
                for c in range(8):
                    ke, ko = _unpack2(kvx[j, pl.ds(hb + c * 16, 16)])
                    pa.append(qlo[c] * ke)
                    pb.append(qhi[c] * ko)
                while len(pa) > 1:
                    pa = [pa[i] + pa[i + 1] for i in range(0, len(pa), 2)]
                    pb = [pb[i] + pb[i + 1] for i in range(0, len(pb), 2)]
                mat_v[pl.ds(j * 17, 16)] = pa[0]
                mat_v[pl.ds(MOFF + j * 17, 16)] = pb[0]
                return 0

            lax.fori_loop(0, VS, dbody, 0, unroll=6)
            ds_ = []
            for moff in (0, MOFF):
                g0 = [plsc.load_gather(mat_v, [moff + jb0 + c]) for c in range(16)]
                g1 = [plsc.load_gather(mat_v, [moff + jb1 + c]) for c in range(16)]
                while len(g0) > 1:
                    g0 = [g0[i] + g0[i + 1] for i in range(0, len(g0), 2)]
                    g1 = [g1[i] + g1[i + 1] for i in range(0, len(g1), 2)]
                ds_.append((g0[0], g1[0]))
            for hx, (da, db) in enumerate(ds_):
                d0 = da * w0
                d1 = jnp.where(lane16 >= (VS - 16), neg_inf, db * w1)
                mv = jnp.maximum(d0, d1)
                for k in (8, 4, 2, 1):
                    p_v[pl.ds(hx * VSP, 16)] = mv
                    mv = jnp.maximum(
                        mv, plsc.load_gather(p_v, [hx * VSP + (lane16 ^ k)]))
                p_v[pl.ds(hx * VSP, 16)] = jnp.exp(d0 - mv)
                p_v[pl.ds(hx * VSP + 16, 16)] = jnp.exp(d1 - mv)

            def obody(j, carry):
                ej = plsc.load_gather(p_v, [jnp.full((16,), j, jnp.int32)])
                ek = plsc.load_gather(p_v, [jnp.full((16,), VSP + j, jnp.int32)])
                o = list(carry)
                for c in range(8):
                    ve, vo = _unpack2(kvx[j, pl.ds(HE // 2 + hb + c * 16, 16)])
                    o[c] = o[c] + ej * ve
                    o[8 + c] = o[8 + c] + ek * vo
                o[16] = o[16] + ej
                o[17] = o[17] + ek
                return tuple(o)

            init18 = tuple(zero16 for _ in range(18))
            res = lax.fori_loop(0, VS, obody, init18, unroll=6)
            inva = 1.0 / res[16]
            invb = 1.0 / res[17]
            for c in range(8):
                ox[pl.ds(hb + c * 16, 16)] = res[c] * inva
                ox[pl.ds(ha + c * 16, 16)] = res[8 + c] * invb

    start_row(0, kva, qa, ska, sqa)

    def pair_body(i, _):
        r0 = 2 * i
        r1 = r0 + 1
        start_row(r1, kvb, qb, skb, sqb)
        gather_copy(r0, kva, ska).wait()
        q_copy(r0, qa, sqa).wait()

        @pl.when(i > 0)
        def _():
            out_copy(r0 - 2, oa, soa).wait()
        compute_row(r0, kva, qa, oa)
        out_copy(r0, oa, soa).start()

        @pl.when(i < RPW // 2 - 1)
        def _():
            start_row(r0 + 2, kva, qa, ska, sqa)
        gather_copy(r1, kvb, skb).wait()
        q_copy(r1, qb, sqb).wait()

        @pl.when(i > 0)
        def _():
            out_copy(r1 - 2, ob, sob).wait()
        compute_row(r1, kvb, qb, ob)
        out_copy(r1, ob, sob).start()
        return 0

    lax.fori_loop(0, RPW // 2, pair_body, 0)
    out_copy(RPW - 2, oa, soa).wait()
    out_copy(RPW - 1, ob, sob).wait()


@functools.lru_cache(maxsize=1)
def _sc_attn():
    mesh = plsc.VectorSubcoreMesh(core_axis_name="c", subcore_axis_name="s")
    return pl.kernel(
        _sc_body,
        mesh=mesh,
        compiler_params=pltpu.CompilerParams(needs_layout_passes=False),
        out_type=jax.ShapeDtypeStruct((T * HE,), jnp.float32),
        scratch_types=[
            pltpu.VMEM((RPW * VSP,), jnp.int32),
            pltpu.VMEM((RPW * VSP,), jnp.float32),
            pltpu.VMEM((HE,), jnp.float32),
            pltpu.VMEM((HE,), jnp.float32),
            pltpu.VMEM((VS, HE), jnp.float32),
            pltpu.VMEM((VS, HE), jnp.float32),
            pltpu.VMEM((2 * VS * 17,), jnp.float32),
            pltpu.VMEM((2 * VSP,), jnp.float32),
            pltpu.VMEM((16 * 33,), jnp.float32),
            pltpu.VMEM((HE,), jnp.float32),
            pltpu.VMEM((HE,), jnp.float32),
            pltpu.SemaphoreType.DMA,
            pltpu.SemaphoreType.DMA,
            pltpu.SemaphoreType.DMA,
            pltpu.SemaphoreType.DMA,
            pltpu.SemaphoreType.DMA,
            pltpu.SemaphoreType.DMA,
        ],
    )


def _front_call(x2, Wq, Wk, Wv, Wp1, bp1, Wp2, bp2, glob, locr):
    nb = T // RB
    fixed = lambda i: (0, 0)
    row = lambda i: (i, 0)
    return pl.pallas_call(
        _front,
        grid=(nb,),
        in_specs=[
            pl.BlockSpec((RB, E), row),
            pl.BlockSpec((E, HE), fixed),
            pl.BlockSpec((E, HE), fixed),
            pl.BlockSpec((E, HE), fixed),
            pl.BlockSpec((E, 4 * E), fixed),
            pl.BlockSpec((1, 4 * E), fixed),
            pl.BlockSpec((1, 4 * E), fixed),
            pl.BlockSpec((4 * E, 2 * KG), fixed),
            pl.BlockSpec((1, 2 * KG), fixed),
            pl.BlockSpec((RB, 2 * KG), row),
            pl.BlockSpec((RB, 2 * KG), row),
        ],
        out_specs=[
            pl.BlockSpec((RB, HE), row),
            pl.BlockSpec((RB, HE), row),
            pl.BlockSpec((RB, VSP), row),
            pl.BlockSpec((RB, VSP), row),
        ],
        out_shape=[
            jax.ShapeDtypeStruct((T, HE), jnp.float32),
            jax.ShapeDtypeStruct((T, HE), jnp.float32),
            jax.ShapeDtypeStruct((T, VSP), jnp.int32),
            jax.ShapeDtypeStruct((T, VSP), jnp.float32),
        ],
    )(x2, Wq, Wk, Wv, Wp1[:E], Wp1[E:], bp1.reshape(1, -1), Wp2,
      bp2.reshape(1, -1), glob, locr)


def _outproj_call(o2, Wu, bu):
    nb = T // RB
    return pl.pallas_call(
        _outproj,
        grid=(nb,),
        in_specs=[
            pl.BlockSpec((RB, HE), lambda i: (i, 0)),
            pl.BlockSpec((HE, E), lambda i: (0, 0)),
            pl.BlockSpec((1, E), lambda i: (0, 0)),
        ],
        out_specs=pl.BlockSpec((RB, E), lambda i: (i, 0)),
        out_shape=jax.ShapeDtypeStruct((T, E), jnp.float32),
    )(o2, Wu, bu.reshape(1, -1))


def kernel(x, Wq, Wk, Wv, Wu, bu, Wp1, bp1, Wp2, bp2):
    x2 = x[0]
    g1, g2 = jax.random.split(jax.random.key(42))
    glob = jax.random.randint(g1, (1, T, KG, 2, 1), 0, T).astype(jnp.float32)
    locr = jax.random.randint(g2, (1, T, KG, 2, 1), 0, 64).astype(jnp.float32)
    glob = glob.reshape(T, 2 * KG)
    locr = locr.reshape(T, 2 * KG)
    q, kv, ivp, wp = _front_call(x2, Wq, Wk, Wv, Wp1, bp1, Wp2, bp2, glob, locr)
    out_flat = _sc_attn()(q.reshape(-1), kv, ivp.reshape(-1), wp.reshape(-1))
    y = _outproj_call(out_flat.reshape(T, HE), Wu, bu)
    return y.reshape(1, T, E)


# speedup vs baseline: 1.2578x; 1.2578x over previous
import functools

import jax
import jax.numpy as jnp
from jax import lax
from jax.experimental import pallas as pl
from jax.experimental.pallas import tpu as pltpu
from jax.experimental.pallas import tpu_sc as plsc

T = 2048
E = 128
H = 8
HE = H * E
KG = 4
VS = 24
VSP = 32
RB = 256
MIN_SIGMA, SIGMA_SCALE, MMULT = 0.05, 0.1, 3.0
SIGMA_BOOST, EPS = 2.0, 1e-7
NW = 32
RPW = T // NW


def _dot16(a, b):
    return jnp.dot(a.astype(jnp.bfloat16), b.astype(jnp.bfloat16),
                   preferred_element_type=jnp.float32)


def _softplus(v):
    return jnp.maximum(v, 0.0) + jnp.log1p(jnp.exp(-jnp.abs(v)))


def _front(x_ref, wq_ref, wk_ref, wv_ref, wp1a_ref, wp1b_ref, bp1_ref,
           wp2_ref, bp2_ref, glob_ref, locr_ref,
           q_ref, kv_ref, iv_ref, w_ref):
    blk = pl.program_id(0)
    xb = x_ref[...]
    rows_f = (blk * RB + lax.broadcasted_iota(jnp.int32, (RB, 1), 0)).astype(jnp.float32)
    coords = rows_f * (1.0 / T)
    h1 = _dot16(xb, wp1a_ref[...])
    cterm = (coords.astype(jnp.bfloat16).astype(jnp.float32)
             * wp1b_ref[...].astype(jnp.bfloat16).astype(jnp.float32))
    h1 = jnp.maximum(h1 + cterm + bp1_ref[...], 0.0)
    params = _dot16(h1, wp2_ref[...])
    params = params + bp2_ref[...]
    means = jnp.clip(rows_f - MMULT * _softplus(params[:, :KG]), 0.0, T - 1.0)
    sig = (_softplus(params[:, KG:] + SIGMA_BOOST) + MIN_SIGMA) * (T * SIGMA_SCALE)
    fl = jnp.floor(means)

    parts = []
    for g in range(KG):
        fg = fl[:, g:g + 1]
        parts += [fg, fg + 1.0,
                  glob_ref[:, 2 * g:2 * g + 1], glob_ref[:, 2 * g + 1:2 * g + 2],
                  fg - 32.0 + locr_ref[:, 2 * g:2 * g + 1],
                  fg - 32.0 + locr_ref[:, 2 * g + 1:2 * g + 2]]
    idxf = jnp.clip(jnp.concatenate(parts, axis=1), 0.0, T - 1.0)
    ivi = idxf.astype(jnp.int32)

    lane = lax.broadcasted_iota(jnp.int32, (RB, VS), 1)
    dup = jnp.zeros((RB, VS), jnp.bool_)
    for jp in range(VS - 1):
        dup = dup | ((ivi == ivi[:, jp:jp + 1]) & (lane > jp))
    rows_i = blk * RB + lax.broadcasted_iota(jnp.int32, (RB, VS), 0)
    masked = dup | (ivi > rows_i)

    wacc = jnp.zeros((RB, VS), jnp.float32)
    for g in range(KG):
        diff = idxf - means[:, g:g + 1]
        pr = jnp.exp(-0.5 * diff * diff / (sig[:, g:g + 1] + EPS))
        pr = jnp.where(masked, 0.0, pr)
        wacc = wacc + pr / jnp.sum(pr, axis=1, keepdims=True)

    q_ref[...] = _dot16(xb, wq_ref[...]) * (E ** -0.5)

    def _pack2(m):
        mb = m.astype(jnp.bfloat16)
        lo = lax.bitcast_convert_type(mb[:, :HE // 2], jnp.uint16).astype(jnp.uint32)
        hi = lax.bitcast_convert_type(mb[:, HE // 2:], jnp.uint16).astype(jnp.uint32)
        return lax.bitcast_convert_type(lo | (hi << 16), jnp.float32)

    kv_ref[:, :HE // 2] = _pack2(_dot16(xb, wk_ref[...]))
    kv_ref[:, HE // 2:] = _pack2(_dot16(xb, wv_ref[...]))
    iv_ref[...] = jnp.concatenate(
        [ivi, jnp.zeros((RB, VSP - VS), jnp.int32)], axis=1)
    w_ref[...] = jnp.concatenate(
        [wacc, jnp.zeros((RB, VSP - VS), jnp.float32)], axis=1)


def _outproj(o_ref, wu_ref, bu_ref, y_ref):
    y_ref[...] = _dot16(o_ref[...], wu_ref[...]) + bu_ref[...]


def _sc_body(q_hbm, kv_hbm, iv_hbm, w_hbm, out_hbm,
             iv_all, w_all, qa, qb, kva, kvb, mat_v, p_v, erep_v, oa, ob,
             ska, skb, sqa, sqb, soa, sob):
    wid = lax.axis_index("s") * 2 + lax.axis_index("c")
    base = wid * RPW

    pltpu.sync_copy(iv_hbm.at[pl.ds(base * VSP, RPW * VSP)], iv_all)
    pltpu.sync_copy(w_hbm.at[pl.ds(base * VSP, RPW * VSP)], w_all)

    neg_inf = jnp.float32(-jnp.inf)
    lane16 = lax.iota(jnp.int32, 16)
    zero16 = jnp.zeros((16,), jnp.float32)
    j0 = lane16
    j1 = jnp.minimum(lane16 + 16, VS - 1)

    def gather_copy(rl, kvx, sem):
        return pltpu.make_async_copy(
            kv_hbm.at[iv_all.at[pl.ds(rl * VSP, VS)]], kvx, sem)

    def q_copy(rl, qx, sem):
        return pltpu.make_async_copy(
            q_hbm.at[pl.ds((base + rl) * HE, HE)], qx, sem)

    def out_copy(rl, ox, sem):
        return pltpu.make_async_copy(
            ox, out_hbm.at[pl.ds((base + rl) * HE, HE)], sem)

    def start_row(rl, kvx, qx, skx, sqx):
        gather_copy(rl, kvx, skx).start()
        q_copy(rl, qx, sqx).start()

    jb0 = j0 * 17
    jb1 = j1 * 17

    MOFF = VS * 17

    def _unpack2(v16):
        return plsc.unpack(plsc.bitcast(v16, jnp.bfloat16),
                           format=plsc.PackFormat.INTERLEAVED)

    def compute_row(rl, kvx, qx, ox):
        w0 = w_all[pl.ds(rl * VSP, 16)]
        w1 = w_all[pl.ds(rl * VSP + 16, 16)]
        for hp in range(H // 2):
            hb = hp * E
            ha = (hp + 4) * E
            qlo = [qx[pl.ds(hb + c * 16, 16)] for c in range(8)]
            qhi = [qx[pl.ds(ha + c * 16, 16)] for c in range(8)]

            def dbody(j, _):
                pa, pb = [], []
                for c in range(8):
                    ke, ko = _unpack2(kvx[j, pl.ds(hb + c * 16, 16)])
                    pa.append(qlo[c] * ke)
                    pb.append(qhi[c] * ko)
                while len(pa) > 1:
                    pa = [pa[i] + pa[i + 1] for i in range(0, len(pa), 2)]
                    pb = [pb[i] + pb[i + 1] for i in range(0, len(pb), 2)]
                mat_v[pl.ds(j * 17, 16)] = pa[0]
                mat_v[pl.ds(MOFF + j * 17, 16)] = pb[0]
                return 0

            lax.fori_loop(0, VS, dbody, 0, unroll=4)
            ds_ = []
            for moff in (0, MOFF):
                g0 = [plsc.load_gather(mat_v, [moff + jb0 + c]) for c in range(16)]
                g1 = [plsc.load_gather(mat_v, [moff + jb1 + c]) for c in range(16)]
                while len(g0) > 1:
                    g0 = [g0[i] + g0[i + 1] for i in range(0, len(g0), 2)]
                    g1 = [g1[i] + g1[i + 1] for i in range(0, len(g1), 2)]
                ds_.append((g0[0], g1[0]))
            for hx, (da, db) in enumerate(ds_):
                d0 = da * w0
                d1 = jnp.where(lane16 >= (VS - 16), neg_inf, db * w1)
                mv = jnp.maximum(d0, d1)
                for k in (8, 4, 2, 1):
                    p_v[pl.ds(hx * VSP, 16)] = mv
                    mv = jnp.maximum(
                        mv, plsc.load_gather(p_v, [hx * VSP + (lane16 ^ k)]))
                p_v[pl.ds(hx * VSP, 16)] = jnp.exp(d0 - mv)
                p_v[pl.ds(hx * VSP + 16, 16)] = jnp.exp(d1 - mv)

            def obody(j, carry):
                ej = plsc.load_gather(p_v, [jnp.full((16,), j, jnp.int32)])
                ek = plsc.load_gather(p_v, [jnp.full((16,), VSP + j, jnp.int32)])
                o = list(carry)
                for c in range(8):
                    ve, vo = _unpack2(kvx[j, pl.ds(HE // 2 + hb + c * 16, 16)])
                    o[c] = o[c] + ej * ve
                    o[8 + c] = o[8 + c] + ek * vo
                o[16] = o[16] + ej
                o[17] = o[17] + ek
                return tuple(o)

            init18 = tuple(zero16 for _ in range(18))
            res = lax.fori_loop(0, VS, obody, init18, unroll=4)
            inva = 1.0 / res[16]
            invb = 1.0 / res[17]
            for c in range(8):
                ox[pl.ds(hb + c * 16, 16)] = res[c] * inva
                ox[pl.ds(ha + c * 16, 16)] = res[8 + c] * invb

    start_row(0, kva, qa, ska, sqa)

    def pair_body(i, _):
        r0 = 2 * i
        r1 = r0 + 1
        start_row(r1, kvb, qb, skb, sqb)
        gather_copy(r0, kva, ska).wait()
        q_copy(r0, qa, sqa).wait()

        @pl.when(i > 0)
        def _():
            out_copy(r0 - 2, oa, soa).wait()
        compute_row(r0, kva, qa, oa)
        out_copy(r0, oa, soa).start()

        @pl.when(i < RPW // 2 - 1)
        def _():
            start_row(r0 + 2, kva, qa, ska, sqa)
        gather_copy(r1, kvb, skb).wait()
        q_copy(r1, qb, sqb).wait()

        @pl.when(i > 0)
        def _():
            out_copy(r1 - 2, ob, sob).wait()
        compute_row(r1, kvb, qb, ob)
        out_copy(r1, ob, sob).start()
        return 0

    lax.fori_loop(0, RPW // 2, pair_body, 0)
    out_copy(RPW - 2, oa, soa).wait()
    out_copy(RPW - 1, ob, sob).wait()


@functools.lru_cache(maxsize=1)
def _sc_attn():
    mesh = plsc.VectorSubcoreMesh(core_axis_name="c", subcore_axis_name="s")
    return pl.kernel(
        _sc_body,
        mesh=mesh,
        compiler_params=pltpu.CompilerParams(needs_layout_passes=False),
        out_type=jax.ShapeDtypeStruct((T * HE,), jnp.float32),
        scratch_types=[
            pltpu.VMEM((RPW * VSP,), jnp.int32),
            pltpu.VMEM((RPW * VSP,), jnp.float32),
            pltpu.VMEM((HE,), jnp.float32),
            pltpu.VMEM((HE,), jnp.float32),
            pltpu.VMEM((VS, HE), jnp.float32),
            pltpu.VMEM((VS, HE), jnp.float32),
            pltpu.VMEM((2 * VS * 17,), jnp.float32),
            pltpu.VMEM((2 * VSP,), jnp.float32),
            pltpu.VMEM((16 * 33,), jnp.float32),
            pltpu.VMEM((HE,), jnp.float32),
            pltpu.VMEM((HE,), jnp.float32),
            pltpu.SemaphoreType.DMA,
            pltpu.SemaphoreType.DMA,
            pltpu.SemaphoreType.DMA,
            pltpu.SemaphoreType.DMA,
            pltpu.SemaphoreType.DMA,
            pltpu.SemaphoreType.DMA,
        ],
    )


def _front_call(x2, Wq, Wk, Wv, Wp1, bp1, Wp2, bp2, glob, locr):
    nb = T // RB
    fixed = lambda i: (0, 0)
    row = lambda i: (i, 0)
    return pl.pallas_call(
        _front,
        grid=(nb,),
        in_specs=[
            pl.BlockSpec((RB, E), row),
            pl.BlockSpec((E, HE), fixed),
            pl.BlockSpec((E, HE), fixed),
            pl.BlockSpec((E, HE), fixed),
            pl.BlockSpec((E, 4 * E), fixed),
            pl.BlockSpec((1, 4 * E), fixed),
            pl.BlockSpec((1, 4 * E), fixed),
            pl.BlockSpec((4 * E, 2 * KG), fixed),
            pl.BlockSpec((1, 2 * KG), fixed),
            pl.BlockSpec((RB, 2 * KG), row),
            pl.BlockSpec((RB, 2 * KG), row),
        ],
        out_specs=[
            pl.BlockSpec((RB, HE), row),
            pl.BlockSpec((RB, HE), row),
            pl.BlockSpec((RB, VSP), row),
            pl.BlockSpec((RB, VSP), row),
        ],
        out_shape=[
            jax.ShapeDtypeStruct((T, HE), jnp.float32),
            jax.ShapeDtypeStruct((T, HE), jnp.float32),
            jax.ShapeDtypeStruct((T, VSP), jnp.int32),
            jax.ShapeDtypeStruct((T, VSP), jnp.float32),
        ],
    )(x2, Wq, Wk, Wv, Wp1[:E], Wp1[E:], bp1.reshape(1, -1), Wp2,
      bp2.reshape(1, -1), glob, locr)


def _outproj_call(o2, Wu, bu):
    nb = T // RB
    return pl.pallas_call(
        _outproj,
        grid=(nb,),
        in_specs=[
            pl.BlockSpec((RB, HE), lambda i: (i, 0)),
            pl.BlockSpec((HE, E), lambda i: (0, 0)),
            pl.BlockSpec((1, E), lambda i: (0, 0)),
        ],
        out_specs=pl.BlockSpec((RB, E), lambda i: (i, 0)),
        out_shape=jax.ShapeDtypeStruct((T, E), jnp.float32),
    )(o2, Wu, bu.reshape(1, -1))


def kernel(x, Wq, Wk, Wv, Wu, bu, Wp1, bp1, Wp2, bp2):
    x2 = x[0]
    g1, g2 = jax.random.split(jax.random.key(42))
    glob = jax.random.randint(g1, (1, T, KG, 2, 1), 0, T).astype(jnp.float32)
    locr = jax.random.randint(g2, (1, T, KG, 2, 1), 0, 64).astype(jnp.float32)
    glob = glob.reshape(T, 2 * KG)
    locr = locr.reshape(T, 2 * KG)
    q, kv, ivp, wp = _front_call(x2, Wq, Wk, Wv, Wp1, bp1, Wp2, bp2, glob, locr)
    out_flat = _sc_attn()(q.reshape(-1), kv, ivp.reshape(-1), wp.reshape(-1))
    y = _outproj_call(out_flat.reshape(T, HE), Wu, bu)
    return y.reshape(1, T, E)


# speedup vs baseline: 1.3664x; 1.0864x over previous
import functools

import jax
import jax.numpy as jnp
from jax import lax
from jax.experimental import pallas as pl
from jax.experimental.pallas import tpu as pltpu
from jax.experimental.pallas import tpu_sc as plsc

T = 2048
E = 128
H = 8
HE = H * E
KG = 4
VS = 24
VSP = 32
RB = 256
MIN_SIGMA, SIGMA_SCALE, MMULT = 0.05, 0.1, 3.0
SIGMA_BOOST, EPS = 2.0, 1e-7
NW = 32
RPW = T // NW


def _dot16(a, b):
    return jnp.dot(a.astype(jnp.bfloat16), b.astype(jnp.bfloat16),
                   preferred_element_type=jnp.float32)


def _softplus(v):
    return jnp.maximum(v, 0.0) + jnp.log1p(jnp.exp(-jnp.abs(v)))


def _front(x_ref, wq_ref, wk_ref, wv_ref, wp1a_ref, wp1b_ref, bp1_ref,
           wp2_ref, bp2_ref, glob_ref, locr_ref,
           q_ref, kv_ref, iv_ref, w_ref):
    blk = pl.program_id(0)
    xb = x_ref[...]
    rows_f = (blk * RB + lax.broadcasted_iota(jnp.int32, (RB, 1), 0)).astype(jnp.float32)
    coords = rows_f * (1.0 / T)
    h1 = _dot16(xb, wp1a_ref[...])
    cterm = (coords.astype(jnp.bfloat16).astype(jnp.float32)
             * wp1b_ref[...].astype(jnp.bfloat16).astype(jnp.float32))
    h1 = jnp.maximum(h1 + cterm + bp1_ref[...], 0.0)
    params = _dot16(h1, wp2_ref[...])
    params = params + bp2_ref[...]
    means = jnp.clip(rows_f - MMULT * _softplus(params[:, :KG]), 0.0, T - 1.0)
    sig = (_softplus(params[:, KG:] + SIGMA_BOOST) + MIN_SIGMA) * (T * SIGMA_SCALE)
    fl = jnp.floor(means)

    parts = []
    for g in range(KG):
        fg = fl[:, g:g + 1]
        parts += [fg, fg + 1.0,
                  glob_ref[:, 2 * g:2 * g + 1], glob_ref[:, 2 * g + 1:2 * g + 2],
                  fg - 32.0 + locr_ref[:, 2 * g:2 * g + 1],
                  fg - 32.0 + locr_ref[:, 2 * g + 1:2 * g + 2]]
    idxf = jnp.clip(jnp.concatenate(parts, axis=1), 0.0, T - 1.0)
    ivi = idxf.astype(jnp.int32)

    lane = lax.broadcasted_iota(jnp.int32, (RB, VS), 1)
    dup = jnp.zeros((RB, VS), jnp.bool_)
    for jp in range(VS - 1):
        dup = dup | ((ivi == ivi[:, jp:jp + 1]) & (lane > jp))
    rows_i = blk * RB + lax.broadcasted_iota(jnp.int32, (RB, VS), 0)
    masked = dup | (ivi > rows_i)

    wacc = jnp.zeros((RB, VS), jnp.float32)
    for g in range(KG):
        diff = idxf - means[:, g:g + 1]
        pr = jnp.exp(-0.5 * diff * diff / (sig[:, g:g + 1] + EPS))
        pr = jnp.where(masked, 0.0, pr)
        wacc = wacc + pr / jnp.sum(pr, axis=1, keepdims=True)

    q_ref[...] = _dot16(xb, wq_ref[...]) * (E ** -0.5)

    def _pack2(m):
        mb = m.astype(jnp.bfloat16)
        lo = lax.bitcast_convert_type(mb[:, :HE // 2], jnp.uint16).astype(jnp.uint32)
        hi = lax.bitcast_convert_type(mb[:, HE // 2:], jnp.uint16).astype(jnp.uint32)
        return lax.bitcast_convert_type(lo | (hi << 16), jnp.float32)

    kv_ref[:, :HE // 2] = _pack2(_dot16(xb, wk_ref[...]))
    kv_ref[:, HE // 2:] = _pack2(_dot16(xb, wv_ref[...]))
    iv_ref[...] = jnp.concatenate(
        [ivi, jnp.zeros((RB, VSP - VS), jnp.int32)], axis=1)
    w_ref[...] = jnp.concatenate(
        [wacc, jnp.zeros((RB, VSP - VS), jnp.float32)], axis=1)


def _outproj(o_ref, wu_ref, bu_ref, y_ref):
    y_ref[...] = _dot16(o_ref[...], wu_ref[...]) + bu_ref[...]


def _sc_body(q_hbm, kv_hbm, iv_hbm, w_hbm, out_hbm,
             iv_all, w_all, qa, qb, kva, kvb, mat_v, p_v, erep_v, oa, ob,
             ska, skb, sqa, sqb, soa, sob):
    wid = lax.axis_index("s") * 2 + lax.axis_index("c")
    base = wid * RPW

    pltpu.sync_copy(iv_hbm.at[pl.ds(base * VSP, RPW * VSP)], iv_all)
    pltpu.sync_copy(w_hbm.at[pl.ds(base * VSP, RPW * VSP)], w_all)

    neg_inf = jnp.float32(-jnp.inf)
    lane16 = lax.iota(jnp.int32, 16)
    zero16 = jnp.zeros((16,), jnp.float32)
    j0 = lane16
    j1 = jnp.minimum(lane16 + 16, VS - 1)

    def gather_copy(rl, kvx, sem):
        return pltpu.make_async_copy(
            kv_hbm.at[iv_all.at[pl.ds(rl * VSP, VS)]], kvx, sem)

    def q_copy(rl, qx, sem):
        return pltpu.make_async_copy(
            q_hbm.at[pl.ds((base + rl) * HE, HE)], qx, sem)

    def out_copy(rl, ox, sem):
        return pltpu.make_async_copy(
            ox, out_hbm.at[pl.ds((base + rl) * HE, HE)], sem)

    def start_row(rl, kvx, qx, skx, sqx):
        gather_copy(rl, kvx, skx).start()
        q_copy(rl, qx, sqx).start()

    jb0 = j0 * 17
    jb1 = j1 * 17

    MOFF = VS * 17

    def _unpack2(v16):
        return plsc.unpack(plsc.bitcast(v16, jnp.bfloat16),
                           format=plsc.PackFormat.INTERLEAVED)

    def compute_row(rl, kvx, qx, ox):
        w0 = w_all[pl.ds(rl * VSP, 16)]
        w1 = w_all[pl.ds(rl * VSP + 16, 16)]
        for hp in range(H // 2):
            hb = hp * E
            ha = (hp + 4) * E
            qlo = [qx[pl.ds(hb + c * 16, 16)] for c in range(8)]
            qhi = [qx[pl.ds(ha + c * 16, 16)] for c in range(8)]

            def dbody(j, _):
                pa, pb = [], []
                for c in range(8):
                    ke, ko = _unpack2(kvx[j, pl.ds(hb + c * 16, 16)])
                    pa.append(qlo[c] * ke)
                    pb.append(qhi[c] * ko)
                while len(pa) > 1:
                    pa = [pa[i] + pa[i + 1] for i in range(0, len(pa), 2)]
                    pb = [pb[i] + pb[i + 1] for i in range(0, len(pb), 2)]
                mat_v[pl.ds(j * 17, 16)] = pa[0]
                mat_v[pl.ds(MOFF + j * 17, 16)] = pb[0]
                return 0

            lax.fori_loop(0, VS, dbody, 0, unroll=3)
            ds_ = []
            for moff in (0, MOFF):
                g0 = [plsc.load_gather(mat_v, [moff + jb0 + c]) for c in range(16)]
                g1 = [plsc.load_gather(mat_v, [moff + jb1 + c]) for c in range(16)]
                while len(g0) > 1:
                    g0 = [g0[i] + g0[i + 1] for i in range(0, len(g0), 2)]
                    g1 = [g1[i] + g1[i + 1] for i in range(0, len(g1), 2)]
                ds_.append((g0[0], g1[0]))
            for hx, (da, db) in enumerate(ds_):
                d0 = da * w0
                d1 = jnp.where(lane16 >= (VS - 16), neg_inf, db * w1)
                mv = jnp.maximum(d0, d1)
                for k in (8, 4, 2, 1):
                    p_v[pl.ds(hx * VSP, 16)] = mv
                    mv = jnp.maximum(
                        mv, plsc.load_gather(p_v, [hx * VSP + (lane16 ^ k)]))
                p_v[pl.ds(hx * VSP, 16)] = jnp.exp(d0 - mv)
                p_v[pl.ds(hx * VSP + 16, 16)] = jnp.exp(d1 - mv)

            def obody(j, carry):
                ej = plsc.load_gather(p_v, [jnp.full((16,), j, jnp.int32)])
                ek = plsc.load_gather(p_v, [jnp.full((16,), VSP + j, jnp.int32)])
                o = list(carry)
                for c in range(8):
                    ve, vo = _unpack2(kvx[j, pl.ds(HE // 2 + hb + c * 16, 16)])
                    o[c] = o[c] + ej * ve
                    o[8 + c] = o[8 + c] + ek * vo
                o[16] = o[16] + ej
                o[17] = o[17] + ek
                return tuple(o)

            init18 = tuple(zero16 for _ in range(18))
            res = lax.fori_loop(0, VS, obody, init18, unroll=3)
            inva = 1.0 / res[16]
            invb = 1.0 / res[17]
            for c in range(8):
                ox[pl.ds(hb + c * 16, 16)] = res[c] * inva
                ox[pl.ds(ha + c * 16, 16)] = res[8 + c] * invb

    start_row(0, kva, qa, ska, sqa)

    def pair_body(i, _):
        r0 = 2 * i
        r1 = r0 + 1
        start_row(r1, kvb, qb, skb, sqb)
        gather_copy(r0, kva, ska).wait()
        q_copy(r0, qa, sqa).wait()

        @pl.when(i > 0)
        def _():
            out_copy(r0 - 2, oa, soa).wait()
        compute_row(r0, kva, qa, oa)
        out_copy(r0, oa, soa).start()

        @pl.when(i < RPW // 2 - 1)
        def _():
            start_row(r0 + 2, kva, qa, ska, sqa)
        gather_copy(r1, kvb, skb).wait()
        q_copy(r1, qb, sqb).wait()

        @pl.when(i > 0)
        def _():
            out_copy(r1 - 2, ob, sob).wait()
        compute_row(r1, kvb, qb, ob)
        out_copy(r1, ob, sob).start()
        return 0

    lax.fori_loop(0, RPW // 2, pair_body, 0)
    out_copy(RPW - 2, oa, soa).wait()
    out_copy(RPW - 1, ob, sob).wait()


@functools.lru_cache(maxsize=1)
def _sc_attn():
    mesh = plsc.VectorSubcoreMesh(core_axis_name="c", subcore_axis_name="s")
    return pl.kernel(
        _sc_body,
        mesh=mesh,
        compiler_params=pltpu.CompilerParams(needs_layout_passes=False),
        out_type=jax.ShapeDtypeStruct((T * HE,), jnp.float32),
        scratch_types=[
            pltpu.VMEM((RPW * VSP,), jnp.int32),
            pltpu.VMEM((RPW * VSP,), jnp.float32),
            pltpu.VMEM((HE,), jnp.float32),
            pltpu.VMEM((HE,), jnp.float32),
            pltpu.VMEM((VS, HE), jnp.float32),
            pltpu.VMEM((VS, HE), jnp.float32),
            pltpu.VMEM((2 * VS * 17,), jnp.float32),
            pltpu.VMEM((2 * VSP,), jnp.float32),
            pltpu.VMEM((16 * 33,), jnp.float32),
            pltpu.VMEM((HE,), jnp.float32),
            pltpu.VMEM((HE,), jnp.float32),
            pltpu.SemaphoreType.DMA,
            pltpu.SemaphoreType.DMA,
            pltpu.SemaphoreType.DMA,
            pltpu.SemaphoreType.DMA,
            pltpu.SemaphoreType.DMA,
            pltpu.SemaphoreType.DMA,
        ],
    )


def _front_call(x2, Wq, Wk, Wv, Wp1, bp1, Wp2, bp2, glob, locr):
    nb = T // RB
    fixed = lambda i: (0, 0)
    row = lambda i: (i, 0)
    return pl.pallas_call(
        _front,
        grid=(nb,),
        in_specs=[
            pl.BlockSpec((RB, E), row),
            pl.BlockSpec((E, HE), fixed),
            pl.BlockSpec((E, HE), fixed),
            pl.BlockSpec((E, HE), fixed),
            pl.BlockSpec((E, 4 * E), fixed),
            pl.BlockSpec((1, 4 * E), fixed),
            pl.BlockSpec((1, 4 * E), fixed),
            pl.BlockSpec((4 * E, 2 * KG), fixed),
            pl.BlockSpec((1, 2 * KG), fixed),
            pl.BlockSpec((RB, 2 * KG), row),
            pl.BlockSpec((RB, 2 * KG), row),
        ],
        out_specs=[
            pl.BlockSpec((RB, HE), row),
            pl.BlockSpec((RB, HE), row),
            pl.BlockSpec((RB, VSP), row),
            pl.BlockSpec((RB, VSP), row),
        ],
        out_shape=[
            jax.ShapeDtypeStruct((T, HE), jnp.float32),
            jax.ShapeDtypeStruct((T, HE), jnp.float32),
            jax.ShapeDtypeStruct((T, VSP), jnp.int32),
            jax.ShapeDtypeStruct((T, VSP), jnp.float32),
        ],
    )(x2, Wq, Wk, Wv, Wp1[:E], Wp1[E:], bp1.reshape(1, -1), Wp2,
      bp2.reshape(1, -1), glob, locr)


def _outproj_call(o2, Wu, bu):
    nb = T // RB
    return pl.pallas_call(
        _outproj,
        grid=(nb,),
        in_specs=[
            pl.BlockSpec((RB, HE), lambda i: (i, 0)),
            pl.BlockSpec((HE, E), lambda i: (0, 0)),
            pl.BlockSpec((1, E), lambda i: (0, 0)),
        ],
        out_specs=pl.BlockSpec((RB, E), lambda i: (i, 0)),
        out_shape=jax.ShapeDtypeStruct((T, E), jnp.float32),
    )(o2, Wu, bu.reshape(1, -1))


def kernel(x, Wq, Wk, Wv, Wu, bu, Wp1, bp1, Wp2, bp2):
    x2 = x[0]
    g1, g2 = jax.random.split(jax.random.key(42))
    glob = jax.random.randint(g1, (1, T, KG, 2, 1), 0, T).astype(jnp.float32)
    locr = jax.random.randint(g2, (1, T, KG, 2, 1), 0, 64).astype(jnp.float32)
    glob = glob.reshape(T, 2 * KG)
    locr = locr.reshape(T, 2 * KG)
    q, kv, ivp, wp = _front_call(x2, Wq, Wk, Wv, Wp1, bp1, Wp2, bp2, glob, locr)
    out_flat = _sc_attn()(q.reshape(-1), kv, ivp.reshape(-1), wp.reshape(-1))
    y = _outproj_call(out_flat.reshape(T, HE), Wu, bu)
    return y.reshape(1, T, E)
